# baseline (speedup 1.0000x reference)
import functools

import jax
import jax.numpy as jnp
from jax import lax
from jax.experimental import pallas as pl
from jax.experimental.pallas import tpu as pltpu
from jax.experimental.pallas import tpu_sc as plsc

B = 16384
D = 64
NC = 2
NS = 16
L = 16
NW = NC * NS
BPW = B // NW
TILES = BPW // L

_mesh = plsc.VectorSubcoreMesh(
    core_axis_name="c", subcore_axis_name="s", num_cores=NC, num_subcores=NS
)


@functools.partial(
    pl.kernel,
    out_type=jax.ShapeDtypeStruct((B,), jnp.float32),
    mesh=_mesh,
    scratch_types=[
        pltpu.VMEM((BPW,), jnp.int32),
        pltpu.VMEM((BPW,), jnp.int32),
        pltpu.VMEM((BPW, D), jnp.float32),
        pltpu.VMEM((BPW, D), jnp.float32),
        pltpu.VMEM((BPW,), jnp.float32),
        pltpu.SemaphoreType.DMA,
        pltpu.SemaphoreType.DMA,
    ],
    compiler_params=pltpu.CompilerParams(
        use_tc_tiling_on_sc=False, needs_layout_passes=False
    ),
)
def _dot_kernel(uidx_hbm, bidx_hbm, utab_hbm, btab_hbm, out_hbm,
                uidx_v, bidx_v, urows_v, brows_v, out_v, sem_u, sem_b):
    wid = lax.axis_index("s") * NC + lax.axis_index("c")
    base = wid * BPW

    pltpu.sync_copy(uidx_hbm.at[pl.ds(base, BPW)], uidx_v)
    pltpu.sync_copy(bidx_hbm.at[pl.ds(base, BPW)], bidx_v)
    cu = pltpu.async_copy(utab_hbm.at[uidx_v], urows_v, sem_u)
    cb = pltpu.async_copy(btab_hbm.at[bidx_v], brows_v, sem_b)
    cu.wait()
    cb.wait()

    lanes = lax.iota(jnp.int32, L)

    def tile_body(t, _):
        rows = t * L + lanes

        def d_body(d, acc):
            dcol = jnp.full((L,), d, jnp.int32)
            u = plsc.load_gather(urows_v, [rows, dcol])
            b = plsc.load_gather(brows_v, [rows, dcol])
            return acc + u * b

        acc = lax.fori_loop(0, D, d_body, jnp.zeros((L,), jnp.float32),
                            unroll=4)
        out_v[pl.ds(t * L, L)] = acc
        return 0

    lax.fori_loop(0, TILES, tile_body, 0)
    pltpu.sync_copy(out_v, out_hbm.at[pl.ds(base, BPW)])


def kernel(inputs, user_table, book_table):
    user_idx = inputs[:, 1]
    book_idx = inputs[:, 0]
    out = _dot_kernel(user_idx, book_idx, user_table, book_table)
    return out.reshape(B, 1)


# speedup vs baseline: 1.5417x; 1.5417x over previous
import functools
import jax
import jax.numpy as jnp
from jax import lax
from jax.experimental import pallas as pl
from jax.experimental.pallas import tpu as pltpu
from jax.experimental.pallas import tpu_sc as plsc

B = 16384
D = 64
NC, NS, L = 2, 16, 16
NW = NC * NS
BPW = B // NW

_mesh = plsc.VectorSubcoreMesh(core_axis_name="c", subcore_axis_name="s",
                               num_cores=NC, num_subcores=NS)

CH = 128

@functools.partial(
    pl.kernel,
    out_type=jax.ShapeDtypeStruct((B,), jnp.float32),
    mesh=_mesh,
    scratch_types=[
        pltpu.VMEM((BPW,), jnp.int32),
        pltpu.VMEM((BPW,), jnp.int32),
        pltpu.VMEM((CH, D), jnp.float32),
        pltpu.VMEM((CH, D), jnp.float32),
        pltpu.VMEM((BPW,), jnp.float32),
        pltpu.SemaphoreType.DMA,
        pltpu.SemaphoreType.DMA,
    ],
    compiler_params=pltpu.CompilerParams(
        use_tc_tiling_on_sc=True, needs_layout_passes=False
    ),
)
def _dot_kernel(uidx_hbm, bidx_hbm, utab_hbm, btab_hbm, out_hbm,
                uidx_v, bidx_v, urows_v, brows_v, out_v, sem_u, sem_b):
    wid = lax.axis_index("s") * NC + lax.axis_index("c")
    base = wid * BPW

    pltpu.sync_copy(uidx_hbm.at[pl.ds(base, BPW)], uidx_v)
    pltpu.sync_copy(bidx_hbm.at[pl.ds(base, BPW)], bidx_v)

    lanes = lax.iota(jnp.int32, L)

    def chunk_body(c, _):
        c0 = c * CH

        def issue_body(g, _):
            uvec = uidx_v[pl.ds(c0 + g * L, L)]
            bvec = bidx_v[pl.ds(c0 + g * L, L)]
            for i in range(L):
                pltpu.async_copy(
                    utab_hbm.at[pl.ds(uvec[i], 1)],
                    urows_v.at[pl.ds(g * L + i, 1)], sem_u)
                pltpu.async_copy(
                    btab_hbm.at[pl.ds(bvec[i], 1)],
                    brows_v.at[pl.ds(g * L + i, 1)], sem_b)
            return 0

        lax.fori_loop(0, CH // L, issue_body, 0)

        def drain_body(i, _):
            pltpu.make_async_copy(utab_hbm.at[pl.ds(0, 1)], urows_v.at[pl.ds(0, 1)], sem_u).wait()
            pltpu.make_async_copy(btab_hbm.at[pl.ds(0, 1)], brows_v.at[pl.ds(0, 1)], sem_b).wait()
            return 0

        lax.fori_loop(0, CH, drain_body, 0)

        def tile_body(t, _):
            rows = t * L + lanes

            def d_body(d, acc):
                dcol = jnp.full((L,), d, jnp.int32)
                u = plsc.load_gather(urows_v, [rows, dcol])
                b = plsc.load_gather(brows_v, [rows, dcol])
                return acc + u * b

            acc = lax.fori_loop(0, D, d_body, jnp.zeros((L,), jnp.float32),
                                unroll=4)
            out_v[pl.ds(c0 + t * L, L)] = acc
            return 0

        lax.fori_loop(0, CH // L, tile_body, 0)
        return 0

    lax.fori_loop(0, BPW // CH, chunk_body, 0)
    pltpu.sync_copy(out_v, out_hbm.at[pl.ds(base, BPW)])


def kernel(inputs, user_table, book_table):
    user_idx = inputs[:, 1]
    book_idx = inputs[:, 0]
    out = _dot_kernel(user_idx, book_idx, user_table, book_table)
    return out.reshape(B, 1)


# speedup vs baseline: 1.6403x; 1.0639x over previous
import functools

import jax
import jax.numpy as jnp
from jax import lax
from jax.experimental import pallas as pl
from jax.experimental.pallas import tpu as pltpu
from jax.experimental.pallas import tpu_sc as plsc

B = 16384
D = 64
V = 1000000
NC = 2
NS = 16
L = 16
NW = NC * NS
BPW = B // NW
CV = 256
VFULL = 999936
NCH = VFULL // CV
NTAIL = V - VFULL
SLOTS = 16
CAP = B + L

_mesh = plsc.VectorSubcoreMesh(
    core_axis_name="c", subcore_axis_name="s", num_cores=NC, num_subcores=NS
)

_params = pltpu.CompilerParams(
    use_tc_tiling_on_sc=True, needs_layout_passes=False
)


@functools.partial(
    pl.kernel,
    out_type=(
        jax.ShapeDtypeStruct((B * D,), jnp.float32),
        jax.ShapeDtypeStruct((B * D,), jnp.float32),
    ),
    mesh=_mesh,
    scratch_types=[
        pltpu.VMEM((B,), jnp.int32),
        pltpu.VMEM((CAP,), jnp.int32),
        pltpu.VMEM((CAP,), jnp.int32),
        pltpu.VMEM((D, CV), jnp.float32),
        pltpu.VMEM((SLOTS * D,), jnp.float32),
        pltpu.SemaphoreType.DMA,
    ],
    compiler_params=_params,
)
def _gather_kernel(uidx_hbm, bidx_hbm, utab_hbm, btab_hbm,
                   utail_hbm, btail_hbm, uvec_hbm, bvec_hbm,
                   idx_v, ilist_v, klist_v, chunk_v, stage_v, semS):
    wid = lax.axis_index("s") * NC + lax.axis_index("c")
    lanes = lax.iota(jnp.int32, L)
    nch_w = jnp.where(wid < NCH - (NCH // NW) * NW, NCH // NW + 1, NCH // NW)

    for side_idx_hbm, tab_hbm, tail_hbm, vec_hbm in (
        (uidx_hbm, utab_hbm, utail_hbm, uvec_hbm),
        (bidx_hbm, btab_hbm, btail_hbm, bvec_hbm),
    ):
        pltpu.sync_copy(side_idx_hbm, idx_v)

        def bscan(v, off):
            iv = idx_v[pl.ds(v * L, L)]
            m = ((jnp.right_shift(iv, 8) & (NW - 1)) == wid) & (iv < VFULL)
            cs = lax.cumsum(m.astype(jnp.int32))
            pos = off + cs - 1
            plsc.store_scatter(ilist_v, [pos], iv, mask=m)
            plsc.store_scatter(klist_v, [pos], v * L + lanes, mask=m)
            return off + cs[L - 1]

        n_w = lax.fori_loop(0, B // L, bscan, jnp.int32(0))
        nv = jnp.right_shift(n_w + (L - 1), 4)

        def chunk_body(j, n_ent0):
            c = wid + j * NW
            off = pl.multiple_of(c * CV, 128)
            pltpu.sync_copy(tab_hbm.at[:, pl.ds(off, CV)], chunk_v)

            def vbody(vi, n_ent1):
                vb = vi * L
                lvi = ilist_v[pl.ds(vb, L)]
                m0 = (jnp.right_shift(lvi, 8) == c) & ((vb + lanes) < n_w)

                def wcond(carry):
                    m, _ = carry
                    return plsc.all_reduce_population_count(m)[0] > 0

                def wbody(carry):
                    m, ne = carry
                    e = plsc.all_reduce_ffs(m)[0]
                    pos = vb + e
                    idx_s = ilist_v[pl.ds(pos, L)][0]
                    kk = klist_v[pl.ds(pos, L)][0]
                    colv = jnp.full((L,), idx_s & (CV - 1), jnp.int32)
                    s = ne & (SLOTS - 1)

                    @pl.when(ne >= SLOTS)
                    def _():
                        pltpu.make_async_copy(
                            stage_v.at[pl.ds(0, D)],
                            vec_hbm.at[pl.ds(0, D)], semS).wait()

                    for t in range(D // L):
                        dv = t * L + lanes
                        vt = plsc.load_gather(chunk_v, [dv, colv])
                        stage_v[pl.ds(s * D + t * L, L)] = vt
                    pltpu.async_copy(stage_v.at[pl.ds(s * D, D)],
                                     vec_hbm.at[pl.ds(kk * D, D)], semS)
                    return m & (lanes != e), ne + 1

                m1, n_ent2 = lax.while_loop(wcond, wbody, (m0, n_ent1))
                return n_ent2

            return lax.fori_loop(0, nv, vbody, n_ent0)

        n_ent = lax.fori_loop(0, nch_w, chunk_body, jnp.int32(0))

        def drain(i, _):
            pltpu.make_async_copy(stage_v.at[pl.ds(0, D)],
                                  vec_hbm.at[pl.ds(0, D)], semS).wait()
            return 0

        lax.fori_loop(0, jnp.minimum(n_ent, SLOTS), drain, 0)

        def tail_body(vi, _):
            vb = wid * BPW + vi * L
            iv = idx_v[pl.ds(vb, L)]

            def tcond(m):
                return plsc.all_reduce_population_count(m)[0] > 0

            def tbody(m):
                e = plsc.all_reduce_ffs(m)[0]
                pos = vb + e
                idx_s = idx_v[pl.ds(pos, L)][0]
                ti = idx_s - VFULL
                pltpu.sync_copy(tail_hbm.at[pl.ds(ti * D, D)],
                                stage_v.at[pl.ds(0, D)])
                pltpu.sync_copy(stage_v.at[pl.ds(0, D)],
                                vec_hbm.at[pl.ds(pos * D, D)])
                return m & (lanes != e)

            lax.while_loop(tcond, tbody, iv >= VFULL)
            return 0

        lax.fori_loop(0, BPW // L, tail_body, 0)


@functools.partial(
    pl.kernel,
    out_type=jax.ShapeDtypeStruct((B,), jnp.float32),
    mesh=_mesh,
    scratch_types=[
        pltpu.VMEM((BPW * D,), jnp.float32),
        pltpu.VMEM((BPW * D,), jnp.float32),
        pltpu.VMEM((BPW,), jnp.float32),
    ],
    compiler_params=_params,
)
def _dot_kernel(uvec_hbm, bvec_hbm, out_hbm, u_v, b_v, out_v):
    wid = lax.axis_index("s") * NC + lax.axis_index("c")
    base = wid * BPW
    lanes = lax.iota(jnp.int32, L)

    pltpu.sync_copy(uvec_hbm.at[pl.ds(base * D, BPW * D)], u_v)
    pltpu.sync_copy(bvec_hbm.at[pl.ds(base * D, BPW * D)], b_v)

    def tile_body(t, _):
        rows = (t * L + lanes) * D

        def d_body(d, acc):
            dvec = rows + d
            u = plsc.load_gather(u_v, [dvec])
            b = plsc.load_gather(b_v, [dvec])
            return acc + u * b

        acc = lax.fori_loop(0, D, d_body, jnp.zeros((L,), jnp.float32),
                            unroll=4)
        out_v[pl.ds(t * L, L)] = acc
        return 0

    lax.fori_loop(0, BPW // L, tile_body, 0)
    pltpu.sync_copy(out_v, out_hbm.at[pl.ds(base, BPW)])


def kernel(inputs, user_table, book_table):
    user_idx = inputs[:, 1]
    book_idx = inputs[:, 0]
    utail = user_table[VFULL:, :].reshape(-1)
    btail = book_table[VFULL:, :].reshape(-1)
    uvec, bvec = _gather_kernel(user_idx, book_idx,
                                user_table.T, book_table.T, utail, btail)
    out = _dot_kernel(uvec, bvec)
    return out.reshape(B, 1)


# speedup vs baseline: 3.5142x; 2.1425x over previous
import functools

import jax
import jax.numpy as jnp
from jax import lax
from jax.experimental import pallas as pl
from jax.experimental.pallas import tpu as pltpu
from jax.experimental.pallas import tpu_sc as plsc

B = 16384
D = 64
V = 1000000
NC = 2
NS = 16
L = 16
NW = NC * NS
BPW = B // NW
CV = 512
CSH = 9
VFULL = 999936
NCH = VFULL // CV
CPW = 62
NTAIL = V - VFULL
SLOTS = 16
CAP = B + L

_mesh = plsc.VectorSubcoreMesh(
    core_axis_name="c", subcore_axis_name="s", num_cores=NC, num_subcores=NS
)

_params = pltpu.CompilerParams(
    use_tc_tiling_on_sc=True, needs_layout_passes=False
)


@functools.partial(
    pl.kernel,
    out_type=(
        jax.ShapeDtypeStruct((B * D,), jnp.float32),
        jax.ShapeDtypeStruct((B * D,), jnp.float32),
    ),
    mesh=_mesh,
    scratch_types=[
        pltpu.VMEM((B,), jnp.int32),
        pltpu.VMEM((CAP,), jnp.int32),
        pltpu.VMEM((CAP,), jnp.int32),
        pltpu.VMEM((D, CV), jnp.float32),
        pltpu.VMEM((D, CV), jnp.float32),
        pltpu.VMEM((SLOTS * D,), jnp.float32),
        pltpu.SemaphoreType.DMA,
        pltpu.SemaphoreType.DMA,
        pltpu.SemaphoreType.DMA,
    ],
    compiler_params=_params,
)
def _gather_kernel(uidx_hbm, bidx_hbm, utab_hbm, btab_hbm,
                   utail_hbm, btail_hbm, uvec_hbm, bvec_hbm,
                   idx_v, ilist_v, klist_v, chunk0_v, chunk1_v, stage_v,
                   semS, semC0, semC1):
    wid = lax.axis_index("s") * NC + lax.axis_index("c")
    lanes = lax.iota(jnp.int32, L)
    cbufs = [chunk0_v, chunk1_v]
    csems = [semC0, semC1]

    for side_idx_hbm, tab_hbm, tail_hbm, vec_hbm in (
        (uidx_hbm, utab_hbm, utail_hbm, uvec_hbm),
        (bidx_hbm, btab_hbm, btail_hbm, bvec_hbm),
    ):
        pltpu.sync_copy(side_idx_hbm, idx_v)

        def bscan(v, off):
            iv = idx_v[pl.ds(v * L, L)]
            m = ((jnp.right_shift(iv, CSH) & (NW - 1)) == wid) & (iv < VFULL)
            cs = lax.cumsum(m.astype(jnp.int32))
            pos = off + cs - 1
            plsc.store_scatter(ilist_v, [pos], iv, mask=m)
            plsc.store_scatter(klist_v, [pos], v * L + lanes, mask=m)
            return off + cs[L - 1]

        n_w = lax.fori_loop(0, B // L, bscan, jnp.int32(0))
        nv = jnp.right_shift(n_w + (L - 1), 4)

        def scan_chunk(buf, c, n_ent0):
            def vbody(vi, n_ent1):
                vb = vi * L
                lvi = ilist_v[pl.ds(vb, L)]
                m0 = (jnp.right_shift(lvi, CSH) == c) & ((vb + lanes) < n_w)

                def wcond(carry):
                    m, _ = carry
                    return plsc.all_reduce_population_count(m)[0] > 0

                def wbody(carry):
                    m, ne = carry
                    e = plsc.all_reduce_ffs(m)[0]
                    pos = vb + e
                    idx_s = ilist_v[pl.ds(pos, L)][0]
                    kk = klist_v[pl.ds(pos, L)][0]
                    colv = jnp.full((L,), idx_s & (CV - 1), jnp.int32)
                    s = ne & (SLOTS - 1)

                    @pl.when(ne >= SLOTS)
                    def _():
                        pltpu.make_async_copy(
                            stage_v.at[pl.ds(0, D)],
                            vec_hbm.at[pl.ds(0, D)], semS).wait()

                    for t in range(D // L):
                        dv = t * L + lanes
                        vt = plsc.load_gather(buf, [dv, colv])
                        stage_v[pl.ds(s * D + t * L, L)] = vt
                    pltpu.async_copy(stage_v.at[pl.ds(s * D, D)],
                                     vec_hbm.at[pl.ds(kk * D, D)], semS)
                    return m & (lanes != e), ne + 1

                m1, n_ent2 = lax.while_loop(wcond, wbody, (m0, n_ent1))
                return n_ent2

            return lax.fori_loop(0, nv, vbody, n_ent0)

        def chunk_of(j):
            c = jnp.minimum(wid + j * NW, NCH - 1)
            return c, pl.multiple_of(c * CV, 128)

        _, off0 = chunk_of(0)
        pltpu.async_copy(tab_hbm.at[:, pl.ds(off0, CV)], cbufs[0], csems[0])

        def pair_body(cc, n_ent0):
            n = n_ent0
            for p in (0, 1):
                j = cc * 2 + p

                @pl.when(j + 1 < CPW)
                def _():
                    _, offn = chunk_of(j + 1)
                    pltpu.async_copy(tab_hbm.at[:, pl.ds(offn, CV)],
                                     cbufs[1 - p], csems[1 - p])

                pltpu.make_async_copy(tab_hbm.at[:, pl.ds(0, CV)],
                                      cbufs[p], csems[p]).wait()
                c, _ = chunk_of(j)
                n = scan_chunk(cbufs[p], c, n)
            return n

        n_ent = lax.fori_loop(0, CPW // 2, pair_body, jnp.int32(0))

        def drain(i, _):
            pltpu.make_async_copy(stage_v.at[pl.ds(0, D)],
                                  vec_hbm.at[pl.ds(0, D)], semS).wait()
            return 0

        lax.fori_loop(0, jnp.minimum(n_ent, SLOTS), drain, 0)

        def tail_body(vi, _):
            vb = wid * BPW + vi * L
            iv = idx_v[pl.ds(vb, L)]

            def tcond(m):
                return plsc.all_reduce_population_count(m)[0] > 0

            def tbody(m):
                e = plsc.all_reduce_ffs(m)[0]
                pos = vb + e
                idx_s = idx_v[pl.ds(pos, L)][0]
                ti = idx_s - VFULL
                pltpu.sync_copy(tail_hbm.at[pl.ds(ti * D, D)],
                                stage_v.at[pl.ds(0, D)])
                pltpu.sync_copy(stage_v.at[pl.ds(0, D)],
                                vec_hbm.at[pl.ds(pos * D, D)])
                return m & (lanes != e)

            lax.while_loop(tcond, tbody, iv >= VFULL)
            return 0

        lax.fori_loop(0, BPW // L, tail_body, 0)


@functools.partial(
    pl.kernel,
    out_type=jax.ShapeDtypeStruct((B,), jnp.float32),
    mesh=_mesh,
    scratch_types=[
        pltpu.VMEM((BPW * D,), jnp.float32),
        pltpu.VMEM((BPW * D,), jnp.float32),
        pltpu.VMEM((BPW,), jnp.float32),
    ],
    compiler_params=_params,
)
def _dot_kernel(uvec_hbm, bvec_hbm, out_hbm, u_v, b_v, out_v):
    wid = lax.axis_index("s") * NC + lax.axis_index("c")
    base = wid * BPW
    lanes = lax.iota(jnp.int32, L)

    pltpu.sync_copy(uvec_hbm.at[pl.ds(base * D, BPW * D)], u_v)
    pltpu.sync_copy(bvec_hbm.at[pl.ds(base * D, BPW * D)], b_v)

    def tile_body(t, _):
        rows = (t * L + lanes) * D

        def d_body(d, acc):
            dvec = rows + d
            u = plsc.load_gather(u_v, [dvec])
            b = plsc.load_gather(b_v, [dvec])
            return acc + u * b

        acc = lax.fori_loop(0, D, d_body, jnp.zeros((L,), jnp.float32),
                            unroll=4)
        out_v[pl.ds(t * L, L)] = acc
        return 0

    lax.fori_loop(0, BPW // L, tile_body, 0)
    pltpu.sync_copy(out_v, out_hbm.at[pl.ds(base, BPW)])


def kernel(inputs, user_table, book_table):
    user_idx = inputs[:, 1]
    book_idx = inputs[:, 0]
    utail = user_table[VFULL:, :].reshape(-1)
    btail = book_table[VFULL:, :].reshape(-1)
    uvec, bvec = _gather_kernel(user_idx, book_idx,
                                user_table.T, book_table.T, utail, btail)
    out = _dot_kernel(uvec, bvec)
    return out.reshape(B, 1)


# speedup vs baseline: 3.5622x; 1.0137x over previous
import functools

import jax
import jax.numpy as jnp
from jax import lax
from jax.experimental import pallas as pl
from jax.experimental.pallas import tpu as pltpu
from jax.experimental.pallas import tpu_sc as plsc

B = 16384
D = 64
V = 1000000
NC = 2
NS = 16
L = 16
NW = NC * NS
BPW = B // NW
CV = 512
CSH = 9
VFULL = 999936
NCH = VFULL // CV
CPW = 62
NTAIL = V - VFULL
SLOTS = 16
CAP = B + L

_mesh = plsc.VectorSubcoreMesh(
    core_axis_name="c", subcore_axis_name="s", num_cores=NC, num_subcores=NS
)

_params = pltpu.CompilerParams(
    use_tc_tiling_on_sc=True, needs_layout_passes=False
)


@functools.partial(
    pl.kernel,
    out_type=(
        jax.ShapeDtypeStruct((B * D,), jnp.float32),
        jax.ShapeDtypeStruct((B * D,), jnp.float32),
    ),
    mesh=_mesh,
    scratch_types=[
        pltpu.VMEM((B,), jnp.int32),
        pltpu.VMEM((CAP,), jnp.int32),
        pltpu.VMEM((CAP,), jnp.int32),
        pltpu.VMEM((D, CV), jnp.float32),
        pltpu.VMEM((D, CV), jnp.float32),
        pltpu.VMEM((SLOTS * D,), jnp.float32),
        pltpu.SemaphoreType.DMA,
        pltpu.SemaphoreType.DMA,
        pltpu.SemaphoreType.DMA,
    ],
    compiler_params=_params,
)
def _gather_kernel(uidx_hbm, bidx_hbm, utab_hbm, btab_hbm,
                   utail_hbm, btail_hbm, uvec_hbm, bvec_hbm,
                   idx_v, ilist_v, klist_v, chunk0_v, chunk1_v, stage_v,
                   semS, semC0, semC1):
    wid = lax.axis_index("s") * NC + lax.axis_index("c")
    lanes = lax.iota(jnp.int32, L)
    cbufs = [chunk0_v, chunk1_v]
    csems = [semC0, semC1]

    for side_idx_hbm, tab_hbm, tail_hbm, vec_hbm in (
        (uidx_hbm, utab_hbm, utail_hbm, uvec_hbm),
        (bidx_hbm, btab_hbm, btail_hbm, bvec_hbm),
    ):
        off0 = pl.multiple_of(jnp.minimum(wid, NCH - 1) * CV, 128)
        pltpu.async_copy(tab_hbm.at[:, pl.ds(off0, CV)], cbufs[0], csems[0])

        pltpu.sync_copy(side_idx_hbm, idx_v)

        def bscan(v, off):
            iv = idx_v[pl.ds(v * L, L)]
            m = ((jnp.right_shift(iv, CSH) & (NW - 1)) == wid) & (iv < VFULL)
            cs = lax.cumsum(m.astype(jnp.int32))
            pos = off + cs - 1
            plsc.store_scatter(ilist_v, [pos], iv, mask=m)
            plsc.store_scatter(klist_v, [pos], v * L + lanes, mask=m)
            return off + plsc.all_reduce_population_count(m)[0]

        n_w = lax.fori_loop(0, B // L, bscan, jnp.int32(0), unroll=2)
        nv = jnp.right_shift(n_w + (L - 1), 4)

        def scan_chunk(buf, c, n_ent0):
            def vbody(vi, n_ent1):
                vb = vi * L
                lvi = ilist_v[pl.ds(vb, L)]
                m0 = (jnp.right_shift(lvi, CSH) == c) & ((vb + lanes) < n_w)

                def wcond(carry):
                    m, _ = carry
                    return plsc.all_reduce_population_count(m)[0] > 0

                def wbody(carry):
                    m, ne = carry
                    e = plsc.all_reduce_ffs(m)[0]
                    pos = vb + e
                    idx_s = ilist_v[pl.ds(pos, L)][0]
                    kk = klist_v[pl.ds(pos, L)][0]
                    colv = jnp.full((L,), idx_s & (CV - 1), jnp.int32)
                    s = ne & (SLOTS - 1)

                    @pl.when(ne >= SLOTS)
                    def _():
                        pltpu.make_async_copy(
                            stage_v.at[pl.ds(0, D)],
                            vec_hbm.at[pl.ds(0, D)], semS).wait()

                    for t in range(D // L):
                        dv = t * L + lanes
                        vt = plsc.load_gather(buf, [dv, colv])
                        stage_v[pl.ds(s * D + t * L, L)] = vt
                    pltpu.async_copy(stage_v.at[pl.ds(s * D, D)],
                                     vec_hbm.at[pl.ds(kk * D, D)], semS)
                    return m & (lanes != e), ne + 1

                m1, n_ent2 = lax.while_loop(wcond, wbody, (m0, n_ent1))
                return n_ent2

            return lax.fori_loop(0, nv, vbody, n_ent0)

        def chunk_of(j):
            c = jnp.minimum(wid + j * NW, NCH - 1)
            return c, pl.multiple_of(c * CV, 128)

        def pair_body(cc, n_ent0):
            n = n_ent0
            for p in (0, 1):
                j = cc * 2 + p

                @pl.when(j + 1 < CPW)
                def _():
                    _, offn = chunk_of(j + 1)
                    pltpu.async_copy(tab_hbm.at[:, pl.ds(offn, CV)],
                                     cbufs[1 - p], csems[1 - p])

                pltpu.make_async_copy(tab_hbm.at[:, pl.ds(0, CV)],
                                      cbufs[p], csems[p]).wait()
                c, _ = chunk_of(j)
                n = scan_chunk(cbufs[p], c, n)
            return n

        n_ent = lax.fori_loop(0, CPW // 2, pair_body, jnp.int32(0))

        def drain(i, _):
            pltpu.make_async_copy(stage_v.at[pl.ds(0, D)],
                                  vec_hbm.at[pl.ds(0, D)], semS).wait()
            return 0

        lax.fori_loop(0, jnp.minimum(n_ent, SLOTS), drain, 0)

        def tail_body(vi, _):
            vb = wid * BPW + vi * L
            iv = idx_v[pl.ds(vb, L)]

            def tcond(m):
                return plsc.all_reduce_population_count(m)[0] > 0

            def tbody(m):
                e = plsc.all_reduce_ffs(m)[0]
                pos = vb + e
                idx_s = idx_v[pl.ds(pos, L)][0]
                ti = idx_s - VFULL
                pltpu.sync_copy(tail_hbm.at[pl.ds(ti * D, D)],
                                stage_v.at[pl.ds(0, D)])
                pltpu.sync_copy(stage_v.at[pl.ds(0, D)],
                                vec_hbm.at[pl.ds(pos * D, D)])
                return m & (lanes != e)

            lax.while_loop(tcond, tbody, iv >= VFULL)
            return 0

        lax.fori_loop(0, BPW // L, tail_body, 0)


@functools.partial(
    pl.kernel,
    out_type=jax.ShapeDtypeStruct((B,), jnp.float32),
    mesh=_mesh,
    scratch_types=[
        pltpu.VMEM((BPW * D,), jnp.float32),
        pltpu.VMEM((BPW * D,), jnp.float32),
        pltpu.VMEM((BPW,), jnp.float32),
        pltpu.SemaphoreType.DMA,
        pltpu.SemaphoreType.DMA,
    ],
    compiler_params=_params,
)
def _dot_kernel(uvec_hbm, bvec_hbm, out_hbm, u_v, b_v, out_v, semU, semB):
    wid = lax.axis_index("s") * NC + lax.axis_index("c")
    base = wid * BPW
    lanes = lax.iota(jnp.int32, L)

    Q = BPW // 4
    for q in range(4):
        pltpu.async_copy(uvec_hbm.at[pl.ds((base + q * Q) * D, Q * D)],
                         u_v.at[pl.ds(q * Q * D, Q * D)], semU)
        pltpu.async_copy(bvec_hbm.at[pl.ds((base + q * Q) * D, Q * D)],
                         b_v.at[pl.ds(q * Q * D, Q * D)], semB)

    for q in range(4):
        pltpu.make_async_copy(uvec_hbm.at[pl.ds(0, Q * D)],
                              u_v.at[pl.ds(0, Q * D)], semU).wait()
        pltpu.make_async_copy(bvec_hbm.at[pl.ds(0, Q * D)],
                              b_v.at[pl.ds(0, Q * D)], semB).wait()

        def tile_body(t, _):
            rows = (q * Q + t * L + lanes) * D

            def d_body(d, acc):
                dvec = rows + d
                u = plsc.load_gather(u_v, [dvec])
                b = plsc.load_gather(b_v, [dvec])
                return acc + u * b

            acc = lax.fori_loop(0, D, d_body, jnp.zeros((L,), jnp.float32),
                                unroll=4)
            out_v[pl.ds(q * Q + t * L, L)] = acc
            return 0

        lax.fori_loop(0, Q // L, tile_body, 0)
    pltpu.sync_copy(out_v, out_hbm.at[pl.ds(base, BPW)])


def kernel(inputs, user_table, book_table):
    user_idx = inputs[:, 1]
    book_idx = inputs[:, 0]
    utail = user_table[VFULL:, :].reshape(-1)
    btail = book_table[VFULL:, :].reshape(-1)
    uvec, bvec = _gather_kernel(user_idx, book_idx,
                                user_table.T, book_table.T, utail, btail)
    out = _dot_kernel(uvec, bvec)
    return out.reshape(B, 1)
